# baseline (speedup 1.0000x reference)
import functools

import jax
import jax.numpy as jnp
from jax import lax
from jax.experimental import pallas as pl
from jax.experimental.pallas import tpu as pltpu
from jax.experimental.pallas import tpu_sc as plsc

BATCH = 16384
CTX_SIZE = 128

_info = plsc.get_sparse_core_info()
_NC, _NS = _info.num_cores, _info.num_subcores
_NW = _NC * _NS
_ROWS = BATCH // _NW

_mesh = plsc.VectorSubcoreMesh(core_axis_name="c", subcore_axis_name="s")

_NCHUNK = 4
_CROWS = _ROWS // _NCHUNK


@functools.partial(
    pl.kernel,
    mesh=_mesh,
    out_type=[
        jax.ShapeDtypeStruct((BATCH, CTX_SIZE), jnp.float32),
        jax.ShapeDtypeStruct((BATCH, 1), jnp.float32),
    ],
    scratch_types=[
        pltpu.VMEM((_NCHUNK, _CROWS, CTX_SIZE), jnp.float32),
        pltpu.VMEM((_ROWS, 1), jnp.float32),
        pltpu.SemaphoreType.DMA((_NCHUNK,)),
        pltpu.SemaphoreType.DMA((_NCHUNK,)),
        pltpu.SemaphoreType.DMA,
    ],
)
def _fifo_write(cxt_hbm, eps_hbm, out_ctx_hbm, out_eps_hbm,
                bufs, ebuf, gsems, ssems, esem):
    wid = lax.axis_index("s") * _NC + lax.axis_index("c")
    base = wid * _ROWS
    gathers = []
    for j in range(_NCHUNK):
        gathers.append(pltpu.async_copy(
            cxt_hbm.at[pl.ds(base + j * _CROWS, _CROWS)],
            bufs.at[j], gsems.at[j]))
    eg = pltpu.async_copy(eps_hbm.at[pl.ds(base, _ROWS)], ebuf, esem)
    scatters = []
    for j in range(_NCHUNK):
        gathers[j].wait()
        scatters.append(pltpu.async_copy(
            bufs.at[j],
            out_ctx_hbm.at[pl.ds(base + j * _CROWS, _CROWS)],
            ssems.at[j]))
    eg.wait()
    es = pltpu.async_copy(ebuf, out_eps_hbm.at[pl.ds(base, _ROWS)], esem)
    for j in range(_NCHUNK):
        scatters[j].wait()
    es.wait()


def kernel(mem_ctx, mem_eps, cxt, eps):
    out_ctx, out_eps = _fifo_write(cxt, eps)
    return out_ctx, out_eps


# speedup vs baseline: 1.5055x; 1.5055x over previous
import functools

import jax
import jax.numpy as jnp
from jax import lax
from jax.experimental import pallas as pl
from jax.experimental.pallas import tpu as pltpu
from jax.experimental.pallas import tpu_sc as plsc

BATCH = 16384
CTX_SIZE = 128

_info = plsc.get_sparse_core_info()
_NC, _NS = _info.num_cores, _info.num_subcores
_NW = _NC * _NS
_ROWS = BATCH // _NW

_mesh = plsc.VectorSubcoreMesh(core_axis_name="c", subcore_axis_name="s")

_NCHUNK = 8
_CROWS = _ROWS // _NCHUNK


@functools.partial(
    pl.kernel,
    mesh=_mesh,
    out_type=[
        jax.ShapeDtypeStruct((BATCH, CTX_SIZE), jnp.float32),
        jax.ShapeDtypeStruct((BATCH,), jnp.float32),
    ],
    scratch_types=[
        pltpu.VMEM((_NCHUNK, _CROWS, CTX_SIZE), jnp.float32),
        pltpu.VMEM((_ROWS,), jnp.float32),
        pltpu.SemaphoreType.DMA((_NCHUNK,)),
        pltpu.SemaphoreType.DMA((_NCHUNK,)),
        pltpu.SemaphoreType.DMA,
    ],
)
def _fifo_write(cxt_hbm, eps_hbm, out_ctx_hbm, out_eps_hbm,
                bufs, ebuf, gsems, ssems, esem):
    wid = lax.axis_index("s") * _NC + lax.axis_index("c")
    base = wid * _ROWS
    gathers = []
    for j in range(_NCHUNK):
        gathers.append(pltpu.async_copy(
            cxt_hbm.at[pl.ds(base + j * _CROWS, _CROWS)],
            bufs.at[j], gsems.at[j]))
    eg = pltpu.async_copy(eps_hbm.at[pl.ds(base, _ROWS)], ebuf, esem)
    scatters = []
    for j in range(_NCHUNK):
        gathers[j].wait()
        scatters.append(pltpu.async_copy(
            bufs.at[j],
            out_ctx_hbm.at[pl.ds(base + j * _CROWS, _CROWS)],
            ssems.at[j]))
    eg.wait()
    es = pltpu.async_copy(ebuf, out_eps_hbm.at[pl.ds(base, _ROWS)], esem)
    for j in range(_NCHUNK):
        scatters[j].wait()
    es.wait()


def kernel(mem_ctx, mem_eps, cxt, eps):
    out_ctx, out_eps = _fifo_write(cxt, eps.reshape(BATCH))
    return out_ctx, out_eps.reshape(BATCH, 1)


# speedup vs baseline: 1.5280x; 1.0150x over previous
import functools

import jax
import jax.numpy as jnp
from jax import lax
from jax.experimental import pallas as pl
from jax.experimental.pallas import tpu as pltpu
from jax.experimental.pallas import tpu_sc as plsc

BATCH = 16384
CTX_SIZE = 128

_info = plsc.get_sparse_core_info()
_NC, _NS = _info.num_cores, _info.num_subcores
_NW = _NC * _NS
_ROWS = BATCH // _NW

_mesh = plsc.VectorSubcoreMesh(core_axis_name="c", subcore_axis_name="s")

_NCHUNK = 4
_CROWS = _ROWS // _NCHUNK


@functools.partial(
    pl.kernel,
    mesh=_mesh,
    out_type=[
        jax.ShapeDtypeStruct((BATCH, CTX_SIZE), jnp.float32),
        jax.ShapeDtypeStruct((BATCH,), jnp.float32),
    ],
    scratch_types=[
        pltpu.VMEM((_NCHUNK, _CROWS, CTX_SIZE), jnp.float32),
        pltpu.VMEM((_ROWS,), jnp.float32),
        pltpu.SemaphoreType.DMA((_NCHUNK,)),
        pltpu.SemaphoreType.DMA((_NCHUNK,)),
        pltpu.SemaphoreType.DMA,
    ],
)
def _fifo_write(cxt_hbm, eps_hbm, out_ctx_hbm, out_eps_hbm,
                bufs, ebuf, gsems, ssems, esem):
    wid = lax.axis_index("s") * _NC + lax.axis_index("c")
    base = wid * _ROWS
    gathers = []
    for j in range(_NCHUNK):
        gathers.append(pltpu.async_copy(
            cxt_hbm.at[pl.ds(base + j * _CROWS, _CROWS)],
            bufs.at[j], gsems.at[j]))
    eg = pltpu.async_copy(eps_hbm.at[pl.ds(base, _ROWS)], ebuf, esem)
    scatters = []
    for j in range(_NCHUNK):
        gathers[j].wait()
        scatters.append(pltpu.async_copy(
            bufs.at[j],
            out_ctx_hbm.at[pl.ds(base + j * _CROWS, _CROWS)],
            ssems.at[j]))
    eg.wait()
    es = pltpu.async_copy(ebuf, out_eps_hbm.at[pl.ds(base, _ROWS)], esem)
    for j in range(_NCHUNK):
        scatters[j].wait()
    es.wait()


def kernel(mem_ctx, mem_eps, cxt, eps):
    out_ctx, out_eps = _fifo_write(cxt, eps.reshape(BATCH))
    return out_ctx, out_eps.reshape(BATCH, 1)
